# baseline (speedup 1.0000x reference)
import functools

import jax
import jax.numpy as jnp
from jax.experimental import pallas as pl
from jax.experimental.pallas import tpu as pltpu

_K = 512
_INT32_MIN = -2147483648


def _encode_key(x):
    bits = jax.lax.bitcast_convert_type(x, jnp.int32)
    return jnp.where(bits >= 0, bits, jnp.int32(_INT32_MIN) - bits)


def _decode_key(k_int):
    bits = jnp.where(k_int >= 0, k_int, jnp.int32(_INT32_MIN) - k_int)
    return jax.lax.bitcast_convert_type(bits.astype(jnp.int32), jnp.float32)


def _ceil_avg(lo, hi):
    x = lo ^ hi
    return (lo & hi) + (x >> 1) + (x & 1)


def _masked_outer(s_row, t_row, n_rows, n_cols, row_offset):
    v = jnp.reshape(s_row, (n_rows, 1)) * jnp.reshape(t_row, (1, n_cols))
    rows = jax.lax.broadcasted_iota(jnp.int32, (n_rows, n_cols), 0) + row_offset
    cols = jax.lax.broadcasted_iota(jnp.int32, (n_rows, n_cols), 1)
    return jnp.where(rows == cols, jnp.float32(-jnp.inf), v)


def _excl_prefix_axis1(x):
    rows, n = x.shape
    incl = x
    d = 1
    while d < n:
        shifted = jnp.concatenate(
            [jnp.zeros((rows, d), x.dtype), incl[:, : n - d]], axis=1
        )
        incl = incl + shifted
        d *= 2
    return incl - x


def _thresh_kernel(s_ref, t_ref, thr_ref, m_ref, r_ref, v_ref):
    n = t_ref.shape[2]
    s = s_ref[0, 0, :]
    t = t_ref[0, 0, :]
    v = _masked_outer(s, t, n, n, 0)
    v_ref[...] = v
    k_f = jnp.float32(_K)

    a = jnp.max(jnp.abs(s)) * jnp.max(jnp.abs(t))
    key_neg_a = _encode_key(-a)
    key_pos_a = _encode_key(a)

    if n >= _K:
        rowmax = jnp.reshape(jnp.max(v, axis=1), (1, n))
        colmax = jnp.reshape(jnp.max(v, axis=0), (1, n))

        def scond(c):
            lo1, hi1, lo2, hi2 = c
            return jnp.logical_or(lo1 < hi1, lo2 < hi2)

        def sbody(c):
            lo1, hi1, lo2, hi2 = c
            mid1 = _ceil_avg(lo1, hi1)
            mid2 = _ceil_avg(lo2, hi2)
            c1 = jnp.sum((rowmax >= _decode_key(mid1)).astype(jnp.float32))
            c2 = jnp.sum((colmax >= _decode_key(mid2)).astype(jnp.float32))
            ge1 = c1 >= k_f
            ge2 = c2 >= k_f
            return (
                jnp.where(ge1, mid1, lo1),
                jnp.where(ge1, hi1, mid1 - 1),
                jnp.where(ge2, mid2, lo2),
                jnp.where(ge2, hi2, mid2 - 1),
            )

        lo1, _, lo2, _ = jax.lax.while_loop(
            scond, sbody, (key_neg_a, key_pos_a, key_neg_a, key_pos_a)
        )
        seed_lo = jnp.maximum(lo1, lo2)
        seed_hi = jnp.maximum(_encode_key(jnp.max(rowmax)), seed_lo)
    else:
        seed_lo = key_neg_a
        seed_hi = key_pos_a

    def mcond(c):
        lo, hi, cnt_lo, cnt_hi, res, done = c
        return jnp.logical_and(done == 0, lo < hi)

    def mbody(c):
        lo, hi, cnt_lo, cnt_hi, res, done = c
        hit_lo = cnt_lo == k_f
        hit_hi = (k_f - cnt_hi) == jnp.float32(1.0)

        def f_hit_lo(_):
            vlo = _decode_key(lo)
            vv = v_ref[...]
            r = jnp.min(jnp.where(vv >= vlo, vv, jnp.float32(jnp.inf)))
            return (lo, hi, cnt_lo, cnt_hi, r, jnp.int32(1))

        def f_hit_hi(_):
            vhi1 = _decode_key(hi + 1)
            vv = v_ref[...]
            r = jnp.max(jnp.where(vv < vhi1, vv, jnp.float32(-jnp.inf)))
            return (lo, hi, cnt_lo, cnt_hi, r, jnp.int32(1))

        def f_bisect(_):
            mid = _ceil_avg(lo, hi)
            tf = _decode_key(mid)
            cnt = jnp.sum((v_ref[...] >= tf).astype(jnp.float32))
            ge = cnt >= k_f
            return (
                jnp.where(ge, mid, lo),
                jnp.where(ge, hi, mid - 1),
                jnp.where(ge, cnt, cnt_lo),
                jnp.where(ge, cnt_hi, cnt),
                res,
                done,
            )

        idx = jnp.where(hit_lo, 0, jnp.where(hit_hi, 1, 2))
        return jax.lax.switch(idx, [f_hit_lo, f_hit_hi, f_bisect], None)

    lo, hi, cnt_lo, cnt_hi, res, done = jax.lax.while_loop(
        mcond,
        mbody,
        (
            seed_lo,
            seed_hi,
            jnp.float32(n * n),
            jnp.float32(0.0),
            jnp.float32(0.0),
            jnp.int32(0),
        ),
    )
    thr = jnp.where(done == 1, res, _decode_key(lo))
    vv = v_ref[...]
    cnt_gt = jnp.sum((vv > thr).astype(jnp.float32))
    eq_rows = jnp.sum((vv == thr).astype(jnp.float32), axis=1, keepdims=True)
    eq_rows = jnp.reshape(eq_rows, (1, n))
    r_ref[0] = _excl_prefix_axis1(eq_rows)

    thr_ref[0] = jnp.full((1, 1), thr, jnp.float32)
    m_ref[0] = jnp.full((1, 1), k_f - cnt_gt, jnp.float32)


def _write_kernel(s_ref, t_ref, thr_ref, m_ref, r_ref, o_ref):
    br = o_ref.shape[1]
    n = o_ref.shape[2]
    b = pl.program_id(1)
    thr = thr_ref[0, 0, 0]
    m = m_ref[0, 0, 0]
    v = _masked_outer(s_ref[0, 0, :], t_ref[0, 0, :], br, n, b * br)
    gt = (v > thr).astype(jnp.float32)
    eq = (v == thr).astype(jnp.float32)
    o_ref[0] = gt

    @pl.when(jnp.sum(eq) > 0)
    def _():
        pref = _excl_prefix_axis1(eq)
        rank = pref + jnp.reshape(r_ref[0, 0, :], (br, 1))
        o_ref[0] = gt + eq * (rank < m).astype(jnp.float32)


@functools.partial(jax.jit, static_argnums=(2, 3))
def _build_adj_mask(emb_s, emb_t, g, n):
    s = emb_s.reshape(g, 1, n)
    t = emb_t.reshape(g, 1, n)

    vec_spec = pl.BlockSpec((1, 1, n), lambda gi: (gi, 0, 0))
    thr, m, r = pl.pallas_call(
        _thresh_kernel,
        grid=(g,),
        in_specs=[vec_spec, vec_spec],
        out_specs=[
            pl.BlockSpec((1, 1, 1), lambda gi: (gi, 0, 0)),
            pl.BlockSpec((1, 1, 1), lambda gi: (gi, 0, 0)),
            pl.BlockSpec((1, 1, n), lambda gi: (gi, 0, 0)),
        ],
        out_shape=[
            jax.ShapeDtypeStruct((g, 1, 1), jnp.float32),
            jax.ShapeDtypeStruct((g, 1, 1), jnp.float32),
            jax.ShapeDtypeStruct((g, 1, n), jnp.float32),
        ],
        scratch_shapes=[pltpu.VMEM((n, n), jnp.float32)],
    )(s, t)

    br = min(n, 256)
    out = pl.pallas_call(
        _write_kernel,
        grid=(g, n // br),
        in_specs=[
            pl.BlockSpec((1, 1, br), lambda gi, bi: (gi, 0, bi)),
            pl.BlockSpec((1, 1, n), lambda gi, bi: (gi, 0, 0)),
            pl.BlockSpec((1, 1, 1), lambda gi, bi: (gi, 0, 0)),
            pl.BlockSpec((1, 1, 1), lambda gi, bi: (gi, 0, 0)),
            pl.BlockSpec((1, 1, br), lambda gi, bi: (gi, 0, bi)),
        ],
        out_specs=pl.BlockSpec((1, br, n), lambda gi, bi: (gi, bi, 0)),
        out_shape=jax.ShapeDtypeStruct((g, n, n), jnp.float32),
    )(s, t, thr, m, r)
    return out


def kernel(emb_s_dynamic, emb_t_dynamic, emb_s_static, emb_t_static,
           emb_s_icd, emb_t_icd, emb_s_reports, emb_t_reports):
    adj_dynamic = _build_adj_mask(emb_s_dynamic, emb_t_dynamic, 8, 1024)
    adj_static = _build_adj_mask(emb_s_static, emb_t_static, 1, 128)
    adj_icd = _build_adj_mask(emb_s_icd, emb_t_icd, 1, 2048)
    adj_reports = _build_adj_mask(emb_s_reports, emb_t_reports, 1, 768)
    return (adj_dynamic, adj_static, adj_icd, adj_reports)


# speedup vs baseline: 1.1806x; 1.1806x over previous
import functools

import jax
import jax.numpy as jnp
from jax.experimental import pallas as pl
from jax.experimental.pallas import tpu as pltpu

_K = 512
_INT32_MIN = -2147483648
_SEED_ROWS = 16
_SEED_COLS = 2048


def _encode_key(x):
    bits = jax.lax.bitcast_convert_type(x, jnp.int32)
    return jnp.where(bits >= 0, bits, jnp.int32(_INT32_MIN) - bits)


def _decode_key(k_int):
    bits = jnp.where(k_int >= 0, k_int, jnp.int32(_INT32_MIN) - k_int)
    return jax.lax.bitcast_convert_type(bits.astype(jnp.int32), jnp.float32)


def _ceil_avg(lo, hi):
    x = lo ^ hi
    return (lo & hi) + (x >> 1) + (x & 1)


def _masked_outer(s_row, t_row, n_rows, n_cols, row_offset):
    v = jnp.reshape(s_row, (n_rows, 1)) * jnp.reshape(t_row, (1, n_cols))
    rows = jax.lax.broadcasted_iota(jnp.int32, (n_rows, n_cols), 0) + row_offset
    cols = jax.lax.broadcasted_iota(jnp.int32, (n_rows, n_cols), 1)
    return jnp.where(rows == cols, jnp.float32(-jnp.inf), v)


def _excl_prefix_axis1(x):
    rows, n = x.shape
    incl = x
    d = 1
    while d < n:
        shifted = jnp.concatenate(
            [jnp.zeros((rows, d), x.dtype), incl[:, : n - d]], axis=1
        )
        incl = incl + shifted
        d *= 2
    return incl - x


def _seed_kernel(sp_ref, tp_ref, lo_ref, hi_ref):
    rr, cc = _SEED_ROWS, _SEED_COLS
    rows_i = jax.lax.broadcasted_iota(jnp.int32, (rr, cc), 0)
    cols_i = jax.lax.broadcasted_iota(jnp.int32, (rr, cc), 1)
    nreal = jnp.where(
        rows_i < 8, 1024,
        jnp.where(rows_i == 8, 2048, jnp.where(rows_i == 9, 768, 0)),
    )
    valid = cols_i < nreal
    s = sp_ref[...]
    t = tp_ref[...]
    ninf = jnp.float32(-jnp.inf)

    def excl_max(x):
        xm = jnp.where(valid, x, ninf)
        x1 = jnp.max(xm, axis=1, keepdims=True)
        j1 = jnp.min(
            jnp.where(xm == x1, cols_i, jnp.int32(cc)), axis=1, keepdims=True
        )
        x2 = jnp.max(jnp.where(cols_i == j1, ninf, xm), axis=1, keepdims=True)
        return jnp.where(cols_i == j1, x2, x1)

    t_hi = excl_max(t)
    t_lo = -excl_max(-t)
    s_hi = excl_max(s)
    s_lo = -excl_max(-s)

    rowmax = jnp.maximum(s * t_hi, s * t_lo)
    colmax = jnp.maximum(t * s_hi, t * s_lo)

    amax = jnp.max(jnp.where(valid, jnp.abs(s), ninf), axis=1, keepdims=True) \
        * jnp.max(jnp.where(valid, jnp.abs(t), ninf), axis=1, keepdims=True)
    key_lo0 = _encode_key(-amax)
    key_hi0 = _encode_key(amax)
    k_f = jnp.float32(_K)

    def sbody(_, c):
        lo1, hi1, lo2, hi2 = c
        mid1 = _ceil_avg(lo1, hi1)
        mid2 = _ceil_avg(lo2, hi2)
        c1 = jnp.sum(
            jnp.where(
                jnp.logical_and(valid, rowmax >= _decode_key(mid1)), 1.0, 0.0
            ),
            axis=1, keepdims=True,
        )
        c2 = jnp.sum(
            jnp.where(
                jnp.logical_and(valid, colmax >= _decode_key(mid2)), 1.0, 0.0
            ),
            axis=1, keepdims=True,
        )
        ge1 = c1 >= k_f
        ge2 = c2 >= k_f
        return (
            jnp.where(ge1, mid1, lo1),
            jnp.where(ge1, hi1, mid1 - 1),
            jnp.where(ge2, mid2, lo2),
            jnp.where(ge2, hi2, mid2 - 1),
        )

    lo1, _, lo2, _ = jax.lax.fori_loop(
        0, 26, sbody, (key_lo0, key_hi0, key_lo0, key_hi0)
    )
    seed_lo = jnp.maximum(lo1, lo2)
    gmax = jnp.max(jnp.where(valid, rowmax, ninf), axis=1, keepdims=True)
    seed_hi = jnp.maximum(_encode_key(gmax), seed_lo)
    lo_ref[...] = seed_lo
    hi_ref[...] = seed_hi


def _thresh_kernel(seeded, s_ref, t_ref, slo_ref, shi_ref,
                   thr_ref, m_ref, r_ref, v_ref, sti_ref, stf_ref):
    n = t_ref.shape[2]
    s = s_ref[0, 0, :]
    t = t_ref[0, 0, :]
    v = _masked_outer(s, t, n, n, 0)
    v_ref[...] = v
    k_f = jnp.float32(_K)

    if seeded:
        seed_lo = slo_ref[0, 0, 0]
        seed_hi = shi_ref[0, 0, 0]
    else:
        a = jnp.max(jnp.abs(s)) * jnp.max(jnp.abs(t))
        seed_lo = _encode_key(-a)
        seed_hi = _encode_key(a)

    sti_ref[0] = seed_lo
    sti_ref[1] = seed_hi
    sti_ref[2] = jnp.int32(0)
    stf_ref[0] = jnp.float32(n * n)
    stf_ref[1] = jnp.float32(0.0)
    stf_ref[2] = jnp.float32(0.0)

    def mbody(_, carry):
        @pl.when(sti_ref[2] == 0)
        def _():
            lo = sti_ref[0]
            hi = sti_ref[1]
            cnt_lo = stf_ref[0]
            cnt_hi = stf_ref[1]
            conv = lo >= hi
            hit_lo = jnp.logical_and(jnp.logical_not(conv), cnt_lo == k_f)
            hit_hi = jnp.logical_and(
                jnp.logical_not(jnp.logical_or(conv, hit_lo)),
                (k_f - cnt_hi) == jnp.float32(1.0),
            )
            els = jnp.logical_not(
                jnp.logical_or(conv, jnp.logical_or(hit_lo, hit_hi))
            )

            @pl.when(conv)
            def _():
                stf_ref[2] = _decode_key(lo)
                sti_ref[2] = jnp.int32(1)

            @pl.when(hit_lo)
            def _():
                vlo = _decode_key(lo)
                vv = v_ref[...]
                stf_ref[2] = jnp.min(
                    jnp.where(vv >= vlo, vv, jnp.float32(jnp.inf))
                )
                sti_ref[2] = jnp.int32(1)

            @pl.when(hit_hi)
            def _():
                vhi1 = _decode_key(hi + 1)
                vv = v_ref[...]
                stf_ref[2] = jnp.max(
                    jnp.where(vv < vhi1, vv, jnp.float32(-jnp.inf))
                )
                sti_ref[2] = jnp.int32(1)

            @pl.when(els)
            def _():
                mid = _ceil_avg(lo, hi)
                tf = _decode_key(mid)
                cnt = jnp.sum((v_ref[...] >= tf).astype(jnp.float32))
                ge = cnt >= k_f
                sti_ref[0] = jnp.where(ge, mid, lo)
                sti_ref[1] = jnp.where(ge, hi, mid - 1)
                stf_ref[0] = jnp.where(ge, cnt, cnt_lo)
                stf_ref[1] = jnp.where(ge, cnt_hi, cnt)

        return carry

    jax.lax.fori_loop(0, 40, mbody, jnp.int32(0))

    thr = stf_ref[2]
    vv = v_ref[...]
    cnt_gt = jnp.sum((vv > thr).astype(jnp.float32))
    eq_rows = jnp.sum((vv == thr).astype(jnp.float32), axis=1, keepdims=True)
    eq_rows = jnp.reshape(eq_rows, (1, n))
    r_ref[0] = _excl_prefix_axis1(eq_rows)

    thr_ref[0] = jnp.full((1, 1), thr, jnp.float32)
    m_ref[0] = jnp.full((1, 1), k_f - cnt_gt, jnp.float32)


def _write_kernel(s_ref, t_ref, thr_ref, m_ref, r_ref, o_ref):
    br = o_ref.shape[1]
    n = o_ref.shape[2]
    b = pl.program_id(1)
    thr = thr_ref[0, 0, 0]
    m = m_ref[0, 0, 0]
    v = _masked_outer(s_ref[0, 0, :], t_ref[0, 0, :], br, n, b * br)
    gt = (v > thr).astype(jnp.float32)
    eq = (v == thr).astype(jnp.float32)
    o_ref[0] = gt

    @pl.when(jnp.sum(eq) > 0)
    def _():
        pref = _excl_prefix_axis1(eq)
        rank = pref + jnp.reshape(r_ref[0, 0, :], (br, 1))
        o_ref[0] = gt + eq * (rank < m).astype(jnp.float32)


def _compute_seeds(svecs, tvecs):
    def pack(vecs):
        rows = [
            jnp.pad(arr, ((0, 0), (0, _SEED_COLS - arr.shape[1])))
            for arr in vecs
        ]
        packed = jnp.concatenate(rows, axis=0)
        return jnp.pad(packed, ((0, _SEED_ROWS - packed.shape[0]), (0, 0)))

    sp = pack(svecs)
    tp = pack(tvecs)
    return pl.pallas_call(
        _seed_kernel,
        in_specs=[
            pl.BlockSpec((_SEED_ROWS, _SEED_COLS), lambda: (0, 0)),
            pl.BlockSpec((_SEED_ROWS, _SEED_COLS), lambda: (0, 0)),
        ],
        out_specs=[
            pl.BlockSpec((_SEED_ROWS, 1), lambda: (0, 0)),
            pl.BlockSpec((_SEED_ROWS, 1), lambda: (0, 0)),
        ],
        out_shape=[
            jax.ShapeDtypeStruct((_SEED_ROWS, 1), jnp.int32),
            jax.ShapeDtypeStruct((_SEED_ROWS, 1), jnp.int32),
        ],
    )(sp, tp)


def _build_adj_mask(s, t, g, n, seed_lo, seed_hi):
    seeded = seed_lo is not None
    if not seeded:
        seed_lo = jnp.zeros((g, 1, 1), jnp.int32)
        seed_hi = jnp.zeros((g, 1, 1), jnp.int32)
    vec_spec = pl.BlockSpec((1, 1, n), lambda gi: (gi, 0, 0))
    seed_spec = pl.BlockSpec((1, 1, 1), lambda gi: (gi, 0, 0))

    thr, m, r = pl.pallas_call(
        functools.partial(_thresh_kernel, seeded),
        grid=(g,),
        in_specs=[vec_spec, vec_spec, seed_spec, seed_spec],
        out_specs=[
            pl.BlockSpec((1, 1, 1), lambda gi: (gi, 0, 0)),
            pl.BlockSpec((1, 1, 1), lambda gi: (gi, 0, 0)),
            pl.BlockSpec((1, 1, n), lambda gi: (gi, 0, 0)),
        ],
        out_shape=[
            jax.ShapeDtypeStruct((g, 1, 1), jnp.float32),
            jax.ShapeDtypeStruct((g, 1, 1), jnp.float32),
            jax.ShapeDtypeStruct((g, 1, n), jnp.float32),
        ],
        scratch_shapes=[
            pltpu.VMEM((n, n), jnp.float32),
            pltpu.SMEM((4,), jnp.int32),
            pltpu.SMEM((4,), jnp.float32),
        ],
    )(s, t, seed_lo, seed_hi)

    br = min(n, 256)
    out = pl.pallas_call(
        _write_kernel,
        grid=(g, n // br),
        in_specs=[
            pl.BlockSpec((1, 1, br), lambda gi, bi: (gi, 0, bi)),
            pl.BlockSpec((1, 1, n), lambda gi, bi: (gi, 0, 0)),
            pl.BlockSpec((1, 1, 1), lambda gi, bi: (gi, 0, 0)),
            pl.BlockSpec((1, 1, 1), lambda gi, bi: (gi, 0, 0)),
            pl.BlockSpec((1, 1, br), lambda gi, bi: (gi, 0, bi)),
        ],
        out_specs=pl.BlockSpec((1, br, n), lambda gi, bi: (gi, bi, 0)),
        out_shape=jax.ShapeDtypeStruct((g, n, n), jnp.float32),
    )(s, t, thr, m, r)
    return out


@jax.jit
def _run(emb_s_dynamic, emb_t_dynamic, emb_s_static, emb_t_static,
         emb_s_icd, emb_t_icd, emb_s_reports, emb_t_reports):
    s_dyn = emb_s_dynamic.reshape(8, 1024)
    t_dyn = emb_t_dynamic.reshape(8, 1024)
    s_icd = emb_s_icd.reshape(1, 2048)
    t_icd = emb_t_icd.reshape(1, 2048)
    s_rep = emb_s_reports.reshape(1, 768)
    t_rep = emb_t_reports.reshape(1, 768)

    seed_lo, seed_hi = _compute_seeds(
        [s_dyn, s_icd, s_rep], [t_dyn, t_icd, t_rep]
    )

    adj_dynamic = _build_adj_mask(
        s_dyn.reshape(8, 1, 1024), t_dyn.reshape(8, 1, 1024), 8, 1024,
        seed_lo[0:8].reshape(8, 1, 1), seed_hi[0:8].reshape(8, 1, 1))
    adj_static = _build_adj_mask(
        emb_s_static.reshape(1, 1, 128), emb_t_static.reshape(1, 1, 128),
        1, 128, None, None)
    adj_icd = _build_adj_mask(
        s_icd.reshape(1, 1, 2048), t_icd.reshape(1, 1, 2048), 1, 2048,
        seed_lo[8:9].reshape(1, 1, 1), seed_hi[8:9].reshape(1, 1, 1))
    adj_reports = _build_adj_mask(
        s_rep.reshape(1, 1, 768), t_rep.reshape(1, 1, 768), 1, 768,
        seed_lo[9:10].reshape(1, 1, 1), seed_hi[9:10].reshape(1, 1, 1))
    return (adj_dynamic, adj_static, adj_icd, adj_reports)


def kernel(emb_s_dynamic, emb_t_dynamic, emb_s_static, emb_t_static,
           emb_s_icd, emb_t_icd, emb_s_reports, emb_t_reports):
    return _run(emb_s_dynamic, emb_t_dynamic, emb_s_static, emb_t_static,
                emb_s_icd, emb_t_icd, emb_s_reports, emb_t_reports)


# speedup vs baseline: 1.2980x; 1.0995x over previous
import functools

import jax
import jax.numpy as jnp
from jax.experimental import pallas as pl
from jax.experimental.pallas import tpu as pltpu

_K = 512
_INT32_MIN = -2147483648
_SEED_ROWS = 16
_SEED_COLS = 2048


def _encode_key(x):
    bits = jax.lax.bitcast_convert_type(x, jnp.int32)
    return jnp.where(bits >= 0, bits, jnp.int32(_INT32_MIN) - bits)


def _decode_key(k_int):
    bits = jnp.where(k_int >= 0, k_int, jnp.int32(_INT32_MIN) - k_int)
    return jax.lax.bitcast_convert_type(bits.astype(jnp.int32), jnp.float32)


def _ceil_avg(lo, hi):
    x = lo ^ hi
    return (lo & hi) + (x >> 1) + (x & 1)


def _masked_outer(s_row, t_row, n_rows, n_cols, row_offset):
    v = jnp.reshape(s_row, (n_rows, 1)) * jnp.reshape(t_row, (1, n_cols))
    rows = jax.lax.broadcasted_iota(jnp.int32, (n_rows, n_cols), 0) + row_offset
    cols = jax.lax.broadcasted_iota(jnp.int32, (n_rows, n_cols), 1)
    return jnp.where(rows == cols, jnp.float32(-jnp.inf), v)


def _excl_prefix_axis1(x):
    rows, n = x.shape
    incl = x
    d = 1
    while d < n:
        shifted = jnp.concatenate(
            [jnp.zeros((rows, d), x.dtype), incl[:, : n - d]], axis=1
        )
        incl = incl + shifted
        d *= 2
    return incl - x


def _excl_prefix_axis0(x):
    n, cols = x.shape
    incl = x
    d = 1
    while d < n:
        shifted = jnp.concatenate(
            [jnp.zeros((d, cols), x.dtype), incl[: n - d, :]], axis=0
        )
        incl = incl + shifted
        d *= 2
    return incl - x


def _seed_kernel(sp_ref, tp_ref, lo_ref, hi_ref):
    rr, cc = _SEED_ROWS, _SEED_COLS
    rows_i = jax.lax.broadcasted_iota(jnp.int32, (rr, cc), 0)
    cols_i = jax.lax.broadcasted_iota(jnp.int32, (rr, cc), 1)
    nreal = jnp.where(
        rows_i < 8, 1024,
        jnp.where(rows_i == 8, 2048, jnp.where(rows_i == 9, 768, 0)),
    )
    valid = cols_i < nreal
    s = sp_ref[...]
    t = tp_ref[...]
    ninf = jnp.float32(-jnp.inf)

    def excl_max(x):
        xm = jnp.where(valid, x, ninf)
        x1 = jnp.max(xm, axis=1, keepdims=True)
        j1 = jnp.min(
            jnp.where(xm == x1, cols_i, jnp.int32(cc)), axis=1, keepdims=True
        )
        x2 = jnp.max(jnp.where(cols_i == j1, ninf, xm), axis=1, keepdims=True)
        return jnp.where(cols_i == j1, x2, x1)

    t_hi = excl_max(t)
    t_lo = -excl_max(-t)
    s_hi = excl_max(s)
    s_lo = -excl_max(-s)

    rowmax = jnp.maximum(s * t_hi, s * t_lo)
    colmax = jnp.maximum(t * s_hi, t * s_lo)

    amax = jnp.max(jnp.where(valid, jnp.abs(s), ninf), axis=1, keepdims=True) \
        * jnp.max(jnp.where(valid, jnp.abs(t), ninf), axis=1, keepdims=True)
    key_lo0 = _encode_key(-amax)
    key_hi0 = _encode_key(amax)
    k_f = jnp.float32(_K)

    def sbody(_, c):
        lo1, hi1, lo2, hi2 = c
        mid1 = _ceil_avg(lo1, hi1)
        mid2 = _ceil_avg(lo2, hi2)
        c1 = jnp.sum(
            jnp.where(
                jnp.logical_and(valid, rowmax >= _decode_key(mid1)), 1.0, 0.0
            ),
            axis=1, keepdims=True,
        )
        c2 = jnp.sum(
            jnp.where(
                jnp.logical_and(valid, colmax >= _decode_key(mid2)), 1.0, 0.0
            ),
            axis=1, keepdims=True,
        )
        ge1 = c1 >= k_f
        ge2 = c2 >= k_f
        return (
            jnp.where(ge1, mid1, lo1),
            jnp.where(ge1, hi1, mid1 - 1),
            jnp.where(ge2, mid2, lo2),
            jnp.where(ge2, hi2, mid2 - 1),
        )

    lo1, _, lo2, _ = jax.lax.fori_loop(
        0, 26, sbody, (key_lo0, key_hi0, key_lo0, key_hi0)
    )
    seed_lo = jnp.maximum(lo1, lo2)
    gmax = jnp.max(jnp.where(valid, rowmax, ninf), axis=1, keepdims=True)
    seed_hi = jnp.maximum(_encode_key(gmax), seed_lo)
    lo_ref[...] = seed_lo
    hi_ref[...] = seed_hi


def _fused_kernel(seeded, br, s_ref, t_ref, slo_ref, shi_ref,
                  o_ref, v_ref, r_ref, sti_ref, stf_ref):
    n = t_ref.shape[2]
    k_f = jnp.float32(_K)
    j = pl.program_id(1)

    @pl.when(j == 0)
    def _compute_threshold():
        s = s_ref[0, 0, :]
        t = t_ref[0, 0, :]
        v_ref[...] = _masked_outer(s, t, n, n, 0)

        if seeded:
            seed_lo = slo_ref[0, 0, 0]
            seed_hi = shi_ref[0, 0, 0]
        else:
            a = jnp.max(jnp.abs(s)) * jnp.max(jnp.abs(t))
            seed_lo = _encode_key(-a)
            seed_hi = _encode_key(a)

        sti_ref[0] = seed_lo
        sti_ref[1] = seed_hi
        sti_ref[2] = jnp.int32(0)
        stf_ref[0] = jnp.float32(n * n)
        stf_ref[1] = jnp.float32(0.0)
        stf_ref[2] = jnp.float32(0.0)

        def mbody(_, carry):
            @pl.when(sti_ref[2] == 0)
            def _():
                lo = sti_ref[0]
                hi = sti_ref[1]
                cnt_lo = stf_ref[0]
                cnt_hi = stf_ref[1]
                conv = lo >= hi
                hit_lo = jnp.logical_and(jnp.logical_not(conv), cnt_lo == k_f)
                hit_hi = jnp.logical_and(
                    jnp.logical_not(jnp.logical_or(conv, hit_lo)),
                    (k_f - cnt_hi) == jnp.float32(1.0),
                )
                els = jnp.logical_not(
                    jnp.logical_or(conv, jnp.logical_or(hit_lo, hit_hi))
                )

                @pl.when(conv)
                def _():
                    stf_ref[2] = _decode_key(lo)
                    sti_ref[2] = jnp.int32(1)

                @pl.when(hit_lo)
                def _():
                    vlo = _decode_key(lo)
                    vv = v_ref[...]
                    stf_ref[2] = jnp.min(
                        jnp.where(vv >= vlo, vv, jnp.float32(jnp.inf))
                    )
                    sti_ref[2] = jnp.int32(1)

                @pl.when(hit_hi)
                def _():
                    vhi1 = _decode_key(hi + 1)
                    vv = v_ref[...]
                    stf_ref[2] = jnp.max(
                        jnp.where(vv < vhi1, vv, jnp.float32(-jnp.inf))
                    )
                    sti_ref[2] = jnp.int32(1)

                @pl.when(els)
                def _():
                    mid = _ceil_avg(lo, hi)
                    tf = _decode_key(mid)
                    cnt = jnp.sum((v_ref[...] >= tf).astype(jnp.float32))
                    ge = cnt >= k_f
                    sti_ref[0] = jnp.where(ge, mid, lo)
                    sti_ref[1] = jnp.where(ge, hi, mid - 1)
                    stf_ref[0] = jnp.where(ge, cnt, cnt_lo)
                    stf_ref[1] = jnp.where(ge, cnt_hi, cnt)

            return carry

        jax.lax.fori_loop(0, 40, mbody, jnp.int32(0))

        thr = stf_ref[2]
        vv = v_ref[...]
        cnt_gt = jnp.sum((vv > thr).astype(jnp.float32))
        eq_col = jnp.sum((vv == thr).astype(jnp.float32), axis=1, keepdims=True)
        r_ref[...] = _excl_prefix_axis0(eq_col)
        stf_ref[3] = k_f - cnt_gt

    thr = stf_ref[2]
    m = stf_ref[3]
    vb = v_ref[pl.ds(j * br, br), :]
    gt = (vb > thr).astype(jnp.float32)
    eq = (vb == thr).astype(jnp.float32)
    o_ref[0] = gt

    @pl.when(jnp.sum(eq) > 0)
    def _():
        pref = _excl_prefix_axis1(eq)
        rank = pref + r_ref[pl.ds(j * br, br), :]
        o_ref[0] = gt + eq * (rank < m).astype(jnp.float32)


def _compute_seeds(svecs, tvecs):
    def pack(vecs):
        rows = [
            jnp.pad(arr, ((0, 0), (0, _SEED_COLS - arr.shape[1])))
            for arr in vecs
        ]
        packed = jnp.concatenate(rows, axis=0)
        return jnp.pad(packed, ((0, _SEED_ROWS - packed.shape[0]), (0, 0)))

    sp = pack(svecs)
    tp = pack(tvecs)
    return pl.pallas_call(
        _seed_kernel,
        in_specs=[
            pl.BlockSpec((_SEED_ROWS, _SEED_COLS), lambda: (0, 0)),
            pl.BlockSpec((_SEED_ROWS, _SEED_COLS), lambda: (0, 0)),
        ],
        out_specs=[
            pl.BlockSpec((_SEED_ROWS, 1), lambda: (0, 0)),
            pl.BlockSpec((_SEED_ROWS, 1), lambda: (0, 0)),
        ],
        out_shape=[
            jax.ShapeDtypeStruct((_SEED_ROWS, 1), jnp.int32),
            jax.ShapeDtypeStruct((_SEED_ROWS, 1), jnp.int32),
        ],
    )(sp, tp)


def _build_adj_mask(s, t, g, n, seed_lo, seed_hi):
    seeded = seed_lo is not None
    if not seeded:
        seed_lo = jnp.zeros((g, 1, 1), jnp.int32)
        seed_hi = jnp.zeros((g, 1, 1), jnp.int32)
    br = min(n, 256)
    vec_spec = pl.BlockSpec((1, 1, n), lambda gi, bi: (gi, 0, 0))
    seed_spec = pl.BlockSpec((1, 1, 1), lambda gi, bi: (gi, 0, 0))

    out = pl.pallas_call(
        functools.partial(_fused_kernel, seeded, br),
        grid=(g, n // br),
        in_specs=[vec_spec, vec_spec, seed_spec, seed_spec],
        out_specs=pl.BlockSpec((1, br, n), lambda gi, bi: (gi, bi, 0)),
        out_shape=jax.ShapeDtypeStruct((g, n, n), jnp.float32),
        scratch_shapes=[
            pltpu.VMEM((n, n), jnp.float32),
            pltpu.VMEM((n, 1), jnp.float32),
            pltpu.SMEM((4,), jnp.int32),
            pltpu.SMEM((4,), jnp.float32),
        ],
    )(s, t, seed_lo, seed_hi)
    return out


@jax.jit
def _run(emb_s_dynamic, emb_t_dynamic, emb_s_static, emb_t_static,
         emb_s_icd, emb_t_icd, emb_s_reports, emb_t_reports):
    s_dyn = emb_s_dynamic.reshape(8, 1024)
    t_dyn = emb_t_dynamic.reshape(8, 1024)
    s_icd = emb_s_icd.reshape(1, 2048)
    t_icd = emb_t_icd.reshape(1, 2048)
    s_rep = emb_s_reports.reshape(1, 768)
    t_rep = emb_t_reports.reshape(1, 768)

    seed_lo, seed_hi = _compute_seeds(
        [s_dyn, s_icd, s_rep], [t_dyn, t_icd, t_rep]
    )

    adj_dynamic = _build_adj_mask(
        s_dyn.reshape(8, 1, 1024), t_dyn.reshape(8, 1, 1024), 8, 1024,
        seed_lo[0:8].reshape(8, 1, 1), seed_hi[0:8].reshape(8, 1, 1))
    adj_static = _build_adj_mask(
        emb_s_static.reshape(1, 1, 128), emb_t_static.reshape(1, 1, 128),
        1, 128, None, None)
    adj_icd = _build_adj_mask(
        s_icd.reshape(1, 1, 2048), t_icd.reshape(1, 1, 2048), 1, 2048,
        seed_lo[8:9].reshape(1, 1, 1), seed_hi[8:9].reshape(1, 1, 1))
    adj_reports = _build_adj_mask(
        s_rep.reshape(1, 1, 768), t_rep.reshape(1, 1, 768), 1, 768,
        seed_lo[9:10].reshape(1, 1, 1), seed_hi[9:10].reshape(1, 1, 1))
    return (adj_dynamic, adj_static, adj_icd, adj_reports)


def kernel(emb_s_dynamic, emb_t_dynamic, emb_s_static, emb_t_static,
           emb_s_icd, emb_t_icd, emb_s_reports, emb_t_reports):
    return _run(emb_s_dynamic, emb_t_dynamic, emb_s_static, emb_t_static,
                emb_s_icd, emb_t_icd, emb_s_reports, emb_t_reports)


# speedup vs baseline: 1.4601x; 1.1248x over previous
import functools

import jax
import jax.numpy as jnp
from jax.experimental import pallas as pl
from jax.experimental.pallas import tpu as pltpu

_K = 512
_INT32_MIN = -2147483648
_SEED_ROWS = 16
_SEED_COLS = 2048


def _encode_key(x):
    bits = jax.lax.bitcast_convert_type(x, jnp.int32)
    return jnp.where(bits >= 0, bits, jnp.int32(_INT32_MIN) - bits)


def _decode_key(k_int):
    bits = jnp.where(k_int >= 0, k_int, jnp.int32(_INT32_MIN) - k_int)
    return jax.lax.bitcast_convert_type(bits.astype(jnp.int32), jnp.float32)


def _ceil_avg(lo, hi):
    x = lo ^ hi
    return (lo & hi) + (x >> 1) + (x & 1)


def _masked_outer(s_row, t_row, n_rows, n_cols, row_offset):
    v = jnp.reshape(s_row, (n_rows, 1)) * jnp.reshape(t_row, (1, n_cols))
    rows = jax.lax.broadcasted_iota(jnp.int32, (n_rows, n_cols), 0) + row_offset
    cols = jax.lax.broadcasted_iota(jnp.int32, (n_rows, n_cols), 1)
    return jnp.where(rows == cols, jnp.float32(-jnp.inf), v)


def _excl_prefix_axis1(x):
    rows, n = x.shape
    incl = x
    d = 1
    while d < n:
        shifted = jnp.concatenate(
            [jnp.zeros((rows, d), x.dtype), incl[:, : n - d]], axis=1
        )
        incl = incl + shifted
        d *= 2
    return incl - x


def _excl_prefix_axis0(x):
    n, cols = x.shape
    incl = x
    d = 1
    while d < n:
        shifted = jnp.concatenate(
            [jnp.zeros((d, cols), x.dtype), incl[: n - d, :]], axis=0
        )
        incl = incl + shifted
        d *= 2
    return incl - x


def _topk_excl(x, valid, cols_i, kdepth):
    cc = x.shape[1]
    ninf = jnp.float32(-jnp.inf)
    cur = jnp.where(valid, x, ninf)
    tops, idxs = [], []
    for _ in range(kdepth):
        xq = jnp.max(cur, axis=1, keepdims=True)
        jq = jnp.min(
            jnp.where(cur == xq, cols_i, jnp.int32(cc)), axis=1, keepdims=True
        )
        tops.append(xq)
        idxs.append(jq)
        cur = jnp.where(cols_i == jq, ninf, cur)
    out = []
    inset = cols_i == idxs[0]
    for p in range(kdepth - 1):
        out.append(jnp.where(inset, tops[p + 1], tops[p]))
        if p + 1 < kdepth - 1:
            inset = jnp.logical_or(inset, cols_i == idxs[p + 1])
    return out


_R = 4


def _seed_kernel(sp_ref, tp_ref, lo_ref, hi_ref):
    rr, cc = _SEED_ROWS, _SEED_COLS
    rows_i = jax.lax.broadcasted_iota(jnp.int32, (rr, cc), 0)
    cols_i = jax.lax.broadcasted_iota(jnp.int32, (rr, cc), 1)
    nreal = jnp.where(
        rows_i < 8, 1024,
        jnp.where(rows_i == 8, 2048, jnp.where(rows_i == 9, 768, 0)),
    )
    valid = cols_i < nreal
    rows1 = jax.lax.broadcasted_iota(jnp.int32, (rr, 1), 0)
    nreal1 = jnp.where(
        rows1 < 8, 1024,
        jnp.where(rows1 == 8, 2048, jnp.where(rows1 == 9, 768, 0)),
    ).astype(jnp.float32)
    s = sp_ref[...]
    t = tp_ref[...]
    ninf = jnp.float32(-jnp.inf)

    t_tops = _topk_excl(t, valid, cols_i, _R + 1)
    t_bots = [-b for b in _topk_excl(-t, valid, cols_i, _R + 1)]
    s_tops = _topk_excl(s, valid, cols_i, _R + 1)
    s_bots = [-b for b in _topk_excl(-s, valid, cols_i, _R + 1)]
    rowtops = [jnp.maximum(s * a, s * b) for a, b in zip(t_tops, t_bots)]
    coltops = [jnp.maximum(t * a, t * b) for a, b in zip(s_tops, s_bots)]

    rstack = jnp.concatenate(rowtops, axis=1)
    cstack = jnp.concatenate(coltops, axis=1)
    vstack = jnp.concatenate([valid] * _R, axis=1)
    rlast = rowtops[_R - 1]
    clast = coltops[_R - 1]

    def cnt_stack(stack, x):
        return jnp.sum(
            jnp.where(jnp.logical_and(vstack, stack >= x), 1.0, 0.0),
            axis=1, keepdims=True,
        )

    def cnt_one(arr, x):
        return jnp.sum(
            jnp.where(jnp.logical_and(valid, arr >= x), 1.0, 0.0),
            axis=1, keepdims=True,
        )

    amax = jnp.max(jnp.where(valid, jnp.abs(s), ninf), axis=1, keepdims=True) \
        * jnp.max(jnp.where(valid, jnp.abs(t), ninf), axis=1, keepdims=True)
    key_lo0 = _encode_key(-amax)
    key_hi0 = _encode_key(amax)
    k_f = jnp.float32(_K)

    def step(lo, hi, f):
        mid = _ceil_avg(lo, hi)
        ge = f(_decode_key(mid)) >= k_f
        return jnp.where(ge, mid, lo), jnp.where(ge, hi, mid - 1)

    def sbody(_, c):
        loA, hiA, loB, hiB, loC, hiC, loD, hiD = c
        loA, hiA = step(loA, hiA, lambda x: cnt_stack(rstack, x))
        loB, hiB = step(loB, hiB, lambda x: cnt_stack(cstack, x))
        loC, hiC = step(
            loC, hiC,
            lambda x: cnt_stack(rstack, x) + (nreal1 - _R) * cnt_one(rlast, x),
        )
        loD, hiD = step(
            loD, hiD,
            lambda x: cnt_stack(cstack, x) + (nreal1 - _R) * cnt_one(clast, x),
        )
        return (loA, hiA, loB, hiB, loC, hiC, loD, hiD)

    init = (key_lo0, key_hi0) * 4
    loA, _, loB, _, _, hiC, _, hiD = jax.lax.fori_loop(0, 26, sbody, init)
    seed_lo = jnp.maximum(loA, loB)
    seed_hi = jnp.maximum(jnp.minimum(hiC, hiD), seed_lo)
    lo_ref[...] = seed_lo
    hi_ref[...] = seed_hi


def _fused_kernel(seeded, br, s_ref, t_ref, slo_ref, shi_ref,
                  o_ref, v_ref, r_ref, sti_ref, stf_ref):
    n = t_ref.shape[2]
    k_f = jnp.float32(_K)
    j = pl.program_id(1)

    @pl.when(j == 0)
    def _compute_threshold():
        s = s_ref[0, 0, :]
        t = t_ref[0, 0, :]
        v_ref[...] = _masked_outer(s, t, n, n, 0)

        if seeded:
            seed_lo = slo_ref[0, 0, 0]
            seed_hi = shi_ref[0, 0, 0]
        else:
            a = jnp.max(jnp.abs(s)) * jnp.max(jnp.abs(t))
            seed_lo = _encode_key(-a)
            seed_hi = _encode_key(a)

        sti_ref[0] = seed_lo
        sti_ref[1] = seed_hi
        sti_ref[2] = jnp.int32(0)
        stf_ref[0] = jnp.float32(n * n)
        stf_ref[1] = jnp.float32(0.0)
        stf_ref[2] = jnp.float32(0.0)

        def mbody(_, carry):
            @pl.when(sti_ref[2] == 0)
            def _():
                lo = sti_ref[0]
                hi = sti_ref[1]
                cnt_lo = stf_ref[0]
                cnt_hi = stf_ref[1]
                conv = lo >= hi
                hit_lo = jnp.logical_and(jnp.logical_not(conv), cnt_lo == k_f)
                hit_hi = jnp.logical_and(
                    jnp.logical_not(jnp.logical_or(conv, hit_lo)),
                    (k_f - cnt_hi) == jnp.float32(1.0),
                )
                els = jnp.logical_not(
                    jnp.logical_or(conv, jnp.logical_or(hit_lo, hit_hi))
                )

                @pl.when(conv)
                def _():
                    stf_ref[2] = _decode_key(lo)
                    sti_ref[2] = jnp.int32(1)

                @pl.when(hit_lo)
                def _():
                    vlo = _decode_key(lo)
                    vv = v_ref[...]
                    stf_ref[2] = jnp.min(
                        jnp.where(vv >= vlo, vv, jnp.float32(jnp.inf))
                    )
                    sti_ref[2] = jnp.int32(1)

                @pl.when(hit_hi)
                def _():
                    vhi1 = _decode_key(hi + 1)
                    vv = v_ref[...]
                    stf_ref[2] = jnp.max(
                        jnp.where(vv < vhi1, vv, jnp.float32(-jnp.inf))
                    )
                    sti_ref[2] = jnp.int32(1)

                @pl.when(els)
                def _():
                    mid = _ceil_avg(lo, hi)
                    tf = _decode_key(mid)
                    cnt = jnp.sum((v_ref[...] >= tf).astype(jnp.float32))
                    ge = cnt >= k_f
                    sti_ref[0] = jnp.where(ge, mid, lo)
                    sti_ref[1] = jnp.where(ge, hi, mid - 1)
                    stf_ref[0] = jnp.where(ge, cnt, cnt_lo)
                    stf_ref[1] = jnp.where(ge, cnt_hi, cnt)

            return carry

        jax.lax.fori_loop(0, 40, mbody, jnp.int32(0))

        thr = stf_ref[2]
        vv = v_ref[...]
        cnt_gt = jnp.sum((vv > thr).astype(jnp.float32))
        eq_col = jnp.sum((vv == thr).astype(jnp.float32), axis=1, keepdims=True)
        r_ref[...] = _excl_prefix_axis0(eq_col)
        stf_ref[3] = k_f - cnt_gt

    thr = stf_ref[2]
    m = stf_ref[3]
    vb = v_ref[pl.ds(j * br, br), :]
    gt = (vb > thr).astype(jnp.float32)
    eq = (vb == thr).astype(jnp.float32)
    o_ref[0] = gt

    @pl.when(jnp.sum(eq) > 0)
    def _():
        pref = _excl_prefix_axis1(eq)
        rank = pref + r_ref[pl.ds(j * br, br), :]
        o_ref[0] = gt + eq * (rank < m).astype(jnp.float32)


def _compute_seeds(svecs, tvecs):
    def pack(vecs):
        rows = [
            jnp.pad(arr, ((0, 0), (0, _SEED_COLS - arr.shape[1])))
            for arr in vecs
        ]
        packed = jnp.concatenate(rows, axis=0)
        return jnp.pad(packed, ((0, _SEED_ROWS - packed.shape[0]), (0, 0)))

    sp = pack(svecs)
    tp = pack(tvecs)
    return pl.pallas_call(
        _seed_kernel,
        in_specs=[
            pl.BlockSpec((_SEED_ROWS, _SEED_COLS), lambda: (0, 0)),
            pl.BlockSpec((_SEED_ROWS, _SEED_COLS), lambda: (0, 0)),
        ],
        out_specs=[
            pl.BlockSpec((_SEED_ROWS, 1), lambda: (0, 0)),
            pl.BlockSpec((_SEED_ROWS, 1), lambda: (0, 0)),
        ],
        out_shape=[
            jax.ShapeDtypeStruct((_SEED_ROWS, 1), jnp.int32),
            jax.ShapeDtypeStruct((_SEED_ROWS, 1), jnp.int32),
        ],
    )(sp, tp)


def _build_adj_mask(s, t, g, n, seed_lo, seed_hi):
    seeded = seed_lo is not None
    if not seeded:
        seed_lo = jnp.zeros((g, 1, 1), jnp.int32)
        seed_hi = jnp.zeros((g, 1, 1), jnp.int32)
    br = min(n, 256)
    vec_spec = pl.BlockSpec((1, 1, n), lambda gi, bi: (gi, 0, 0))
    seed_spec = pl.BlockSpec((1, 1, 1), lambda gi, bi: (gi, 0, 0))

    out = pl.pallas_call(
        functools.partial(_fused_kernel, seeded, br),
        grid=(g, n // br),
        in_specs=[vec_spec, vec_spec, seed_spec, seed_spec],
        out_specs=pl.BlockSpec((1, br, n), lambda gi, bi: (gi, bi, 0)),
        out_shape=jax.ShapeDtypeStruct((g, n, n), jnp.float32),
        scratch_shapes=[
            pltpu.VMEM((n, n), jnp.float32),
            pltpu.VMEM((n, 1), jnp.float32),
            pltpu.SMEM((4,), jnp.int32),
            pltpu.SMEM((4,), jnp.float32),
        ],
    )(s, t, seed_lo, seed_hi)
    return out


@jax.jit
def _run(emb_s_dynamic, emb_t_dynamic, emb_s_static, emb_t_static,
         emb_s_icd, emb_t_icd, emb_s_reports, emb_t_reports):
    s_dyn = emb_s_dynamic.reshape(8, 1024)
    t_dyn = emb_t_dynamic.reshape(8, 1024)
    s_icd = emb_s_icd.reshape(1, 2048)
    t_icd = emb_t_icd.reshape(1, 2048)
    s_rep = emb_s_reports.reshape(1, 768)
    t_rep = emb_t_reports.reshape(1, 768)

    seed_lo, seed_hi = _compute_seeds(
        [s_dyn, s_icd, s_rep], [t_dyn, t_icd, t_rep]
    )

    adj_dynamic = _build_adj_mask(
        s_dyn.reshape(8, 1, 1024), t_dyn.reshape(8, 1, 1024), 8, 1024,
        seed_lo[0:8].reshape(8, 1, 1), seed_hi[0:8].reshape(8, 1, 1))
    adj_static = _build_adj_mask(
        emb_s_static.reshape(1, 1, 128), emb_t_static.reshape(1, 1, 128),
        1, 128, None, None)
    adj_icd = _build_adj_mask(
        s_icd.reshape(1, 1, 2048), t_icd.reshape(1, 1, 2048), 1, 2048,
        seed_lo[8:9].reshape(1, 1, 1), seed_hi[8:9].reshape(1, 1, 1))
    adj_reports = _build_adj_mask(
        s_rep.reshape(1, 1, 768), t_rep.reshape(1, 1, 768), 1, 768,
        seed_lo[9:10].reshape(1, 1, 1), seed_hi[9:10].reshape(1, 1, 1))
    return (adj_dynamic, adj_static, adj_icd, adj_reports)


def kernel(emb_s_dynamic, emb_t_dynamic, emb_s_static, emb_t_static,
           emb_s_icd, emb_t_icd, emb_s_reports, emb_t_reports):
    return _run(emb_s_dynamic, emb_t_dynamic, emb_s_static, emb_t_static,
                emb_s_icd, emb_t_icd, emb_s_reports, emb_t_reports)
